# baseline (speedup 1.0000x reference)
import jax
import jax.numpy as jnp
from jax import lax
from jax.experimental import pallas as pl
from jax.experimental.pallas import tpu as pltpu
from jax.experimental.pallas import tpu_sc as plsc

DELTA = 0.1
SAMPLE_SIZE = 2500
B, H, W = 16, 512, 512

PAD = 2560
NT = PAD // 16
UNROLL = 4
CHUNK_ROWS = 32
HALF_ROWS = H // 2
NCH = HALF_ROWS // CHUNK_ROWS
NBUF = 6


def _sc_body(dp2, dg2, i0_hbm, i1_hbm, out,
             b0, b1, b2, b3, b4, b5, i0_v, i1_v, vp0, vp1, vg0, vg1,
             s0, s1, s2, s3, s4, s5):
    s = lax.axis_index("s")
    c = lax.axis_index("c")

    pltpu.sync_copy(i0_hbm, i0_v)
    pltpu.sync_copy(i1_hbm, i1_v)

    bufs = [b0, b1, b2, b3, b4, b5]
    sems = [s0, s1, s2, s3, s4, s5]
    steps = []
    for k in range(2 * NCH):
        a, q = k % 2, k // 2
        src = dp2 if a == 0 else dg2
        dst0, dst1 = (vp0, vp1) if a == 0 else (vg0, vg1)
        steps.append((src, q, dst0, dst1))

    def fire(k):
        src, q, _, _ = steps[k]
        row0 = H * s + HALF_ROWS * c + CHUNK_ROWS * q
        return pltpu.async_copy(
            src.at[pl.ds(row0, CHUNK_ROWS), :], bufs[k % NBUF], sems[k % NBUF])

    nsteps = 2 * NCH
    depth = 4
    pending = [fire(k) for k in range(depth)]
    for k in range(nsteps):
        _, q, dst0, dst1 = steps[k]
        buf = bufs[k % NBUF]
        pending[k].wait()
        if k + depth < nsteps:
            pending.append(fire(k + depth))
        base = HALF_ROWS * c + CHUNK_ROWS * q

        def chunk_body(t, carry):
            for u in range(UNROLL):
                sl = pl.ds((t * UNROLL + u) * 16, 16)
                for iv, vv in ((i0_v, dst0), (i1_v, dst1)):
                    idx = iv[sl]
                    r_local = (idx >> 9) - base
                    m = plsc.bitcast(r_local, jnp.uint32) < CHUNK_ROWS
                    lr = jnp.bitwise_and(r_local, CHUNK_ROWS - 1)
                    lc = jnp.bitwise_and(idx, W - 1)
                    v = plsc.load_gather(buf, [lr, lc], mask=m)
                    vv[sl] = jnp.where(m, v, vv[sl])
            return carry

        lax.fori_loop(0, NT // UNROLL, chunk_body, 0)

    w = c * 16 + s
    pltpu.sync_copy(vp0, out.at[w])
    pltpu.sync_copy(vp1, out.at[32 + w])
    pltpu.sync_copy(vg0, out.at[64 + w])
    pltpu.sync_copy(vg1, out.at[96 + w])


def _make_sc_kernel():
    mesh = plsc.VectorSubcoreMesh(core_axis_name="c", subcore_axis_name="s")
    return pl.kernel(
        _sc_body,
        out_type=jax.ShapeDtypeStruct((128, PAD), jnp.float32),
        mesh=mesh,
        compiler_params=pltpu.CompilerParams(needs_layout_passes=False),
        scratch_types=[
            pltpu.VMEM((CHUNK_ROWS, W), jnp.float32),
            pltpu.VMEM((CHUNK_ROWS, W), jnp.float32),
            pltpu.VMEM((CHUNK_ROWS, W), jnp.float32),
            pltpu.VMEM((CHUNK_ROWS, W), jnp.float32),
            pltpu.VMEM((CHUNK_ROWS, W), jnp.float32),
            pltpu.VMEM((CHUNK_ROWS, W), jnp.float32),
            pltpu.VMEM((PAD,), jnp.int32),
            pltpu.VMEM((PAD,), jnp.int32),
            pltpu.VMEM((PAD,), jnp.float32),
            pltpu.VMEM((PAD,), jnp.float32),
            pltpu.VMEM((PAD,), jnp.float32),
            pltpu.VMEM((PAD,), jnp.float32),
            pltpu.SemaphoreType.DMA,
            pltpu.SemaphoreType.DMA,
            pltpu.SemaphoreType.DMA,
            pltpu.SemaphoreType.DMA,
            pltpu.SemaphoreType.DMA,
            pltpu.SemaphoreType.DMA,
        ],
    )


def _loss_body(x_ref, i0_ref, i1_ref, o_ref):
    m0 = jnp.broadcast_to((i0_ref[...] >> 9) < HALF_ROWS, (16, PAD))
    m1 = jnp.broadcast_to((i1_ref[...] >> 9) < HALF_ROWS, (16, PAD))
    p0 = jnp.where(m0, x_ref[0:16], x_ref[16:32])
    p1 = jnp.where(m1, x_ref[32:48], x_ref[48:64])
    g0 = jnp.where(m0, x_ref[64:80], x_ref[80:96])
    g1 = jnp.where(m1, x_ref[96:112], x_ref[112:128])
    d = p0 - p1
    g = g0 - g1
    sq = d * d
    hinge = jnp.maximum(-d * jnp.sign(g), 0.0)
    loss = jnp.where(jnp.abs(g) < DELTA, sq, hinge)
    col = lax.broadcasted_iota(jnp.int32, (16, PAD), 1)
    loss = jnp.where(col < SAMPLE_SIZE, loss, 0.0)
    o_ref[0, 0] = jnp.sum(loss) * (1.0 / (SAMPLE_SIZE * B))


_loss = pl.pallas_call(
    _loss_body,
    out_shape=jax.ShapeDtypeStruct((1, 1), jnp.float32),
    out_specs=pl.BlockSpec(memory_space=pltpu.SMEM),
)


@jax.jit
def kernel(depth_pred, depth_gt, indices):
    dp2 = depth_pred.reshape(B * H, W)
    dg2 = depth_gt.reshape(B * H, W)
    i0 = jnp.pad(indices[:, 0], (0, PAD - SAMPLE_SIZE))
    i1 = jnp.pad(indices[:, 1], (0, PAD - SAMPLE_SIZE))
    vals = _make_sc_kernel()(dp2, dg2, i0, i1)
    return _loss(vals, i0, i1)[0, 0]


# speedup vs baseline: 1.3995x; 1.3995x over previous
import jax
import jax.numpy as jnp
from jax import lax
from jax.experimental import pallas as pl
from jax.experimental.pallas import tpu as pltpu
from jax.experimental.pallas import tpu_sc as plsc

DELTA = 0.1
SAMPLE_SIZE = 2500
B, H, W = 16, 512, 512

PAD = 2560
NT = PAD // 16
UNROLL = 4
CHUNK_ROWS = 64
HALF_ROWS = H // 2
NCH = HALF_ROWS // CHUNK_ROWS
NBUF = 3


def _sc_body(dp2, dg2, i0_hbm, i1_hbm, out,
             b0, b1, b2, i0_v, i1_v, vp0, vp1, vg0, vg1,
             s0, s1, s2):
    s = lax.axis_index("s")
    c = lax.axis_index("c")

    bufs = [b0, b1, b2]
    sems = [s0, s1, s2]
    steps = []
    for k in range(2 * NCH):
        a, q = k % 2, k // 2
        src = dp2 if a == 0 else dg2
        dst0, dst1 = (vp0, vp1) if a == 0 else (vg0, vg1)
        steps.append((src, q, dst0, dst1))

    def fire(k):
        src, q, _, _ = steps[k]
        row0 = H * s + HALF_ROWS * c + CHUNK_ROWS * q
        return pltpu.async_copy(
            src.at[pl.ds(row0, CHUNK_ROWS), :], bufs[k % NBUF], sems[k % NBUF])

    nsteps = 2 * NCH
    pending = [fire(0), fire(1)]
    pltpu.sync_copy(i0_hbm, i0_v)
    pltpu.sync_copy(i1_hbm, i1_v)
    for k in range(nsteps):
        _, q, dst0, dst1 = steps[k]
        buf = bufs[k % NBUF]
        pending[k].wait()
        if k + 2 < nsteps:
            pending.append(fire(k + 2))
        base = HALF_ROWS * c + CHUNK_ROWS * q

        def chunk_body(t, carry):
            for u in range(UNROLL):
                sl = pl.ds((t * UNROLL + u) * 16, 16)
                for iv, vv in ((i0_v, dst0), (i1_v, dst1)):
                    idx = iv[sl]
                    r_local = (idx >> 9) - base
                    m = plsc.bitcast(r_local, jnp.uint32) < CHUNK_ROWS
                    lr = jnp.bitwise_and(r_local, CHUNK_ROWS - 1)
                    lc = jnp.bitwise_and(idx, W - 1)
                    v = plsc.load_gather(buf, [lr, lc], mask=m)
                    vv[sl] = jnp.where(m, v, vv[sl])
            return carry

        lax.fori_loop(0, NT // UNROLL, chunk_body, 0)

    w = c * 16 + s
    pltpu.sync_copy(vp0, out.at[w])
    pltpu.sync_copy(vp1, out.at[32 + w])
    pltpu.sync_copy(vg0, out.at[64 + w])
    pltpu.sync_copy(vg1, out.at[96 + w])


def _make_sc_kernel():
    mesh = plsc.VectorSubcoreMesh(core_axis_name="c", subcore_axis_name="s")
    return pl.kernel(
        _sc_body,
        out_type=jax.ShapeDtypeStruct((128, PAD), jnp.float32),
        mesh=mesh,
        compiler_params=pltpu.CompilerParams(needs_layout_passes=False),
        scratch_types=[
            pltpu.VMEM((CHUNK_ROWS, W), jnp.float32),
            pltpu.VMEM((CHUNK_ROWS, W), jnp.float32),
            pltpu.VMEM((CHUNK_ROWS, W), jnp.float32),
            pltpu.VMEM((PAD,), jnp.int32),
            pltpu.VMEM((PAD,), jnp.int32),
            pltpu.VMEM((PAD,), jnp.float32),
            pltpu.VMEM((PAD,), jnp.float32),
            pltpu.VMEM((PAD,), jnp.float32),
            pltpu.VMEM((PAD,), jnp.float32),
            pltpu.SemaphoreType.DMA,
            pltpu.SemaphoreType.DMA,
            pltpu.SemaphoreType.DMA,
        ],
    )


def _loss_body(x_ref, i0_ref, i1_ref, o_ref):
    m0 = jnp.broadcast_to((i0_ref[...] >> 9) < HALF_ROWS, (16, PAD))
    m1 = jnp.broadcast_to((i1_ref[...] >> 9) < HALF_ROWS, (16, PAD))
    p0 = jnp.where(m0, x_ref[0:16], x_ref[16:32])
    p1 = jnp.where(m1, x_ref[32:48], x_ref[48:64])
    g0 = jnp.where(m0, x_ref[64:80], x_ref[80:96])
    g1 = jnp.where(m1, x_ref[96:112], x_ref[112:128])
    d = p0 - p1
    g = g0 - g1
    sq = d * d
    hinge = jnp.maximum(-d * jnp.sign(g), 0.0)
    loss = jnp.where(jnp.abs(g) < DELTA, sq, hinge)
    col = lax.broadcasted_iota(jnp.int32, (16, PAD), 1)
    loss = jnp.where(col < SAMPLE_SIZE, loss, 0.0)
    o_ref[0, 0] = jnp.sum(loss) * (1.0 / (SAMPLE_SIZE * B))


_loss = pl.pallas_call(
    _loss_body,
    out_shape=jax.ShapeDtypeStruct((1, 1), jnp.float32),
    out_specs=pl.BlockSpec(memory_space=pltpu.SMEM),
)


@jax.jit
def kernel(depth_pred, depth_gt, indices):
    dp2 = depth_pred.reshape(B * H, W)
    dg2 = depth_gt.reshape(B * H, W)
    i0 = jnp.pad(indices[:, 0], (0, PAD - SAMPLE_SIZE))
    i1 = jnp.pad(indices[:, 1], (0, PAD - SAMPLE_SIZE))
    vals = _make_sc_kernel()(dp2, dg2, i0, i1)
    return _loss(vals, i0, i1)[0, 0]


# speedup vs baseline: 1.4150x; 1.0110x over previous
import jax
import jax.numpy as jnp
from jax import lax
from jax.experimental import pallas as pl
from jax.experimental.pallas import tpu as pltpu
from jax.experimental.pallas import tpu_sc as plsc

DELTA = 0.1
SAMPLE_SIZE = 2500
B, H, W = 16, 512, 512

PAD = 2560
NT = PAD // 16
UNROLL = 4
CHUNK_ROWS = 64
HALF_ROWS = H // 2
NCH = HALF_ROWS // CHUNK_ROWS
NBUF = 3


def _sc_body(dp2, dg2, i0_hbm, i1_hbm, out,
             b0, b1, b2, i0_v, i1_v, vp0, vp1, vg0, vg1,
             s0, s1, s2, s_out):
    s = lax.axis_index("s")
    c = lax.axis_index("c")

    bufs = [b0, b1, b2]
    sems = [s0, s1, s2]
    steps = []
    for k in range(2 * NCH):
        a, q = k // NCH, k % NCH
        src = dp2 if a == 0 else dg2
        dst0, dst1 = (vp0, vp1) if a == 0 else (vg0, vg1)
        steps.append((src, q, dst0, dst1))

    def fire(k):
        src, q, _, _ = steps[k]
        row0 = H * s + HALF_ROWS * c + CHUNK_ROWS * q
        return pltpu.async_copy(
            src.at[pl.ds(row0, CHUNK_ROWS), :], bufs[k % NBUF], sems[k % NBUF])

    nsteps = 2 * NCH
    pending = [fire(0), fire(1)]
    pltpu.sync_copy(i0_hbm, i0_v)
    pltpu.sync_copy(i1_hbm, i1_v)
    for k in range(nsteps):
        _, q, dst0, dst1 = steps[k]
        buf = bufs[k % NBUF]
        pending[k].wait()
        if k + 2 < nsteps:
            pending.append(fire(k + 2))
        base = HALF_ROWS * c + CHUNK_ROWS * q

        def chunk_body(t, carry):
            for u in range(UNROLL):
                sl = pl.ds((t * UNROLL + u) * 16, 16)
                for iv, vv in ((i0_v, dst0), (i1_v, dst1)):
                    idx = iv[sl]
                    r_local = (idx >> 9) - base
                    m = plsc.bitcast(r_local, jnp.uint32) < CHUNK_ROWS
                    lr = jnp.bitwise_and(r_local, CHUNK_ROWS - 1)
                    lc = jnp.bitwise_and(idx, W - 1)
                    v = plsc.load_gather(buf, [lr, lc], mask=m)
                    vv[sl] = jnp.where(m, v, vv[sl])
            return carry

        lax.fori_loop(0, NT // UNROLL, chunk_body, 0)

        if k == NCH - 1:
            w = c * 16 + s
            out_p = [pltpu.async_copy(vp0, out.at[w], s_out),
                     pltpu.async_copy(vp1, out.at[32 + w], s_out)]

    w = c * 16 + s
    out_g = [pltpu.async_copy(vg0, out.at[64 + w], s_out),
             pltpu.async_copy(vg1, out.at[96 + w], s_out)]
    for cp in out_p + out_g:
        cp.wait()


def _make_sc_kernel():
    mesh = plsc.VectorSubcoreMesh(core_axis_name="c", subcore_axis_name="s")
    return pl.kernel(
        _sc_body,
        out_type=jax.ShapeDtypeStruct((128, PAD), jnp.float32),
        mesh=mesh,
        compiler_params=pltpu.CompilerParams(needs_layout_passes=False),
        scratch_types=[
            pltpu.VMEM((CHUNK_ROWS, W), jnp.float32),
            pltpu.VMEM((CHUNK_ROWS, W), jnp.float32),
            pltpu.VMEM((CHUNK_ROWS, W), jnp.float32),
            pltpu.VMEM((PAD,), jnp.int32),
            pltpu.VMEM((PAD,), jnp.int32),
            pltpu.VMEM((PAD,), jnp.float32),
            pltpu.VMEM((PAD,), jnp.float32),
            pltpu.VMEM((PAD,), jnp.float32),
            pltpu.VMEM((PAD,), jnp.float32),
            pltpu.SemaphoreType.DMA,
            pltpu.SemaphoreType.DMA,
            pltpu.SemaphoreType.DMA,
            pltpu.SemaphoreType.DMA,
        ],
    )


def _loss_body(x_ref, i0_ref, i1_ref, o_ref):
    m0 = jnp.broadcast_to((i0_ref[...] >> 9) < HALF_ROWS, (16, PAD))
    m1 = jnp.broadcast_to((i1_ref[...] >> 9) < HALF_ROWS, (16, PAD))
    p0 = jnp.where(m0, x_ref[0:16], x_ref[16:32])
    p1 = jnp.where(m1, x_ref[32:48], x_ref[48:64])
    g0 = jnp.where(m0, x_ref[64:80], x_ref[80:96])
    g1 = jnp.where(m1, x_ref[96:112], x_ref[112:128])
    d = p0 - p1
    g = g0 - g1
    sq = d * d
    hinge = jnp.maximum(-d * jnp.sign(g), 0.0)
    loss = jnp.where(jnp.abs(g) < DELTA, sq, hinge)
    col = lax.broadcasted_iota(jnp.int32, (16, PAD), 1)
    loss = jnp.where(col < SAMPLE_SIZE, loss, 0.0)
    o_ref[0, 0] = jnp.sum(loss) * (1.0 / (SAMPLE_SIZE * B))


_loss = pl.pallas_call(
    _loss_body,
    out_shape=jax.ShapeDtypeStruct((1, 1), jnp.float32),
    out_specs=pl.BlockSpec(memory_space=pltpu.SMEM),
)


@jax.jit
def kernel(depth_pred, depth_gt, indices):
    dp2 = depth_pred.reshape(B * H, W)
    dg2 = depth_gt.reshape(B * H, W)
    i0 = jnp.pad(indices[:, 0], (0, PAD - SAMPLE_SIZE))
    i1 = jnp.pad(indices[:, 1], (0, PAD - SAMPLE_SIZE))
    vals = _make_sc_kernel()(dp2, dg2, i0, i1)
    return _loss(vals, i0, i1)[0, 0]


# speedup vs baseline: 1.4631x; 1.0340x over previous
import jax
import jax.numpy as jnp
from jax import lax
from jax.experimental import pallas as pl
from jax.experimental.pallas import tpu as pltpu
from jax.experimental.pallas import tpu_sc as plsc

DELTA = 0.1
SAMPLE_SIZE = 2500
B, H, W = 16, 512, 512

PAD = 2560
NT = PAD // 16
UNROLL = 2
CHUNK_ROWS = 64
HALF_ROWS = H // 2
NCH = HALF_ROWS // CHUNK_ROWS
NBUF = 3


def _sc_body(dp2, dg2, i0_hbm, i1_hbm, out,
             b0, b1, b2, i0_v, i1_v, vp0, vp1, vg0, vg1,
             s0, s1, s2, s_out):
    s = lax.axis_index("s")
    c = lax.axis_index("c")

    bufs = [b0, b1, b2]
    sems = [s0, s1, s2]
    steps = []
    for k in range(2 * NCH):
        a, q = k // NCH, k % NCH
        src = dp2 if a == 0 else dg2
        dst0, dst1 = (vp0, vp1) if a == 0 else (vg0, vg1)
        steps.append((src, q, dst0, dst1))

    def fire(k):
        src, q, _, _ = steps[k]
        row0 = H * s + HALF_ROWS * c + CHUNK_ROWS * q
        return pltpu.async_copy(
            src.at[pl.ds(row0, CHUNK_ROWS), :], bufs[k % NBUF], sems[k % NBUF])

    nsteps = 2 * NCH
    pending = [fire(0), fire(1)]
    pltpu.sync_copy(i0_hbm, i0_v)
    pltpu.sync_copy(i1_hbm, i1_v)
    for k in range(nsteps):
        _, q, dst0, dst1 = steps[k]
        buf = bufs[k % NBUF]
        pending[k].wait()
        if k + 2 < nsteps:
            pending.append(fire(k + 2))
        base = HALF_ROWS * c + CHUNK_ROWS * q

        def chunk_body(t, carry):
            for u in range(UNROLL):
                sl = pl.ds((t * UNROLL + u) * 16, 16)
                for iv, vv in ((i0_v, dst0), (i1_v, dst1)):
                    idx = iv[sl]
                    r_local = (idx >> 9) - base
                    m = plsc.bitcast(r_local, jnp.uint32) < CHUNK_ROWS
                    lr = jnp.bitwise_and(r_local, CHUNK_ROWS - 1)
                    lc = jnp.bitwise_and(idx, W - 1)
                    v = plsc.load_gather(buf, [lr, lc], mask=m)
                    vv[sl] = jnp.where(m, v, vv[sl])
            return carry

        lax.fori_loop(0, NT // UNROLL, chunk_body, 0)

        if k == NCH - 1:
            w = c * 16 + s
            out_p = [pltpu.async_copy(vp0, out.at[w], s_out),
                     pltpu.async_copy(vp1, out.at[32 + w], s_out)]

    w = c * 16 + s
    out_g = [pltpu.async_copy(vg0, out.at[64 + w], s_out),
             pltpu.async_copy(vg1, out.at[96 + w], s_out)]
    for cp in out_p + out_g:
        cp.wait()


def _make_sc_kernel():
    mesh = plsc.VectorSubcoreMesh(core_axis_name="c", subcore_axis_name="s")
    return pl.kernel(
        _sc_body,
        out_type=jax.ShapeDtypeStruct((128, PAD), jnp.float32),
        mesh=mesh,
        compiler_params=pltpu.CompilerParams(needs_layout_passes=False),
        scratch_types=[
            pltpu.VMEM((CHUNK_ROWS, W), jnp.float32),
            pltpu.VMEM((CHUNK_ROWS, W), jnp.float32),
            pltpu.VMEM((CHUNK_ROWS, W), jnp.float32),
            pltpu.VMEM((PAD,), jnp.int32),
            pltpu.VMEM((PAD,), jnp.int32),
            pltpu.VMEM((PAD,), jnp.float32),
            pltpu.VMEM((PAD,), jnp.float32),
            pltpu.VMEM((PAD,), jnp.float32),
            pltpu.VMEM((PAD,), jnp.float32),
            pltpu.SemaphoreType.DMA,
            pltpu.SemaphoreType.DMA,
            pltpu.SemaphoreType.DMA,
            pltpu.SemaphoreType.DMA,
        ],
    )


def _loss_body(x_ref, i0_ref, i1_ref, o_ref):
    m0 = jnp.broadcast_to((i0_ref[...] >> 9) < HALF_ROWS, (16, PAD))
    m1 = jnp.broadcast_to((i1_ref[...] >> 9) < HALF_ROWS, (16, PAD))
    p0 = jnp.where(m0, x_ref[0:16], x_ref[16:32])
    p1 = jnp.where(m1, x_ref[32:48], x_ref[48:64])
    g0 = jnp.where(m0, x_ref[64:80], x_ref[80:96])
    g1 = jnp.where(m1, x_ref[96:112], x_ref[112:128])
    d = p0 - p1
    g = g0 - g1
    sq = d * d
    hinge = jnp.maximum(-d * jnp.sign(g), 0.0)
    loss = jnp.where(jnp.abs(g) < DELTA, sq, hinge)
    col = lax.broadcasted_iota(jnp.int32, (16, PAD), 1)
    loss = jnp.where(col < SAMPLE_SIZE, loss, 0.0)
    o_ref[0, 0] = jnp.sum(loss) * (1.0 / (SAMPLE_SIZE * B))


_loss = pl.pallas_call(
    _loss_body,
    out_shape=jax.ShapeDtypeStruct((1, 1), jnp.float32),
    out_specs=pl.BlockSpec(memory_space=pltpu.SMEM),
)


@jax.jit
def kernel(depth_pred, depth_gt, indices):
    dp2 = depth_pred.reshape(B * H, W)
    dg2 = depth_gt.reshape(B * H, W)
    i0 = jnp.pad(indices[:, 0], (0, PAD - SAMPLE_SIZE))
    i1 = jnp.pad(indices[:, 1], (0, PAD - SAMPLE_SIZE))
    vals = _make_sc_kernel()(dp2, dg2, i0, i1)
    return _loss(vals, i0, i1)[0, 0]


# speedup vs baseline: 1.4826x; 1.0133x over previous
import jax
import jax.numpy as jnp
from jax import lax
from jax.experimental import pallas as pl
from jax.experimental.pallas import tpu as pltpu
from jax.experimental.pallas import tpu_sc as plsc

DELTA = 0.1
SAMPLE_SIZE = 2500
B, H, W = 16, 512, 512

PAD = 2560
NT = PAD // 16
UNROLL = 1
CHUNK_ROWS = 64
HALF_ROWS = H // 2
NCH = HALF_ROWS // CHUNK_ROWS
NBUF = 3


def _sc_body(dp2, dg2, i0_hbm, i1_hbm, out,
             b0, b1, b2, i0_v, i1_v, vp0, vp1, vg0, vg1,
             s0, s1, s2, s_out):
    s = lax.axis_index("s")
    c = lax.axis_index("c")

    bufs = [b0, b1, b2]
    sems = [s0, s1, s2]
    steps = []
    for k in range(2 * NCH):
        a, q = k // NCH, k % NCH
        src = dp2 if a == 0 else dg2
        dst0, dst1 = (vp0, vp1) if a == 0 else (vg0, vg1)
        steps.append((src, q, dst0, dst1))

    def fire(k):
        src, q, _, _ = steps[k]
        row0 = H * s + HALF_ROWS * c + CHUNK_ROWS * q
        return pltpu.async_copy(
            src.at[pl.ds(row0, CHUNK_ROWS), :], bufs[k % NBUF], sems[k % NBUF])

    nsteps = 2 * NCH
    pending = [fire(0), fire(1)]
    pltpu.sync_copy(i0_hbm, i0_v)
    pltpu.sync_copy(i1_hbm, i1_v)
    for k in range(nsteps):
        _, q, dst0, dst1 = steps[k]
        buf = bufs[k % NBUF]
        pending[k].wait()
        if k + 2 < nsteps:
            pending.append(fire(k + 2))
        base = HALF_ROWS * c + CHUNK_ROWS * q

        def chunk_body(t, carry):
            for u in range(UNROLL):
                sl = pl.ds((t * UNROLL + u) * 16, 16)
                for iv, vv in ((i0_v, dst0), (i1_v, dst1)):
                    idx = iv[sl]
                    r_local = (idx >> 9) - base
                    m = plsc.bitcast(r_local, jnp.uint32) < CHUNK_ROWS
                    lr = jnp.bitwise_and(r_local, CHUNK_ROWS - 1)
                    lc = jnp.bitwise_and(idx, W - 1)
                    v = plsc.load_gather(buf, [lr, lc], mask=m)
                    vv[sl] = jnp.where(m, v, vv[sl])
            return carry

        lax.fori_loop(0, NT // UNROLL, chunk_body, 0)

        if k == NCH - 1:
            w = c * 16 + s
            out_p = [pltpu.async_copy(vp0, out.at[w], s_out),
                     pltpu.async_copy(vp1, out.at[32 + w], s_out)]

    w = c * 16 + s
    out_g = [pltpu.async_copy(vg0, out.at[64 + w], s_out),
             pltpu.async_copy(vg1, out.at[96 + w], s_out)]
    for cp in out_p + out_g:
        cp.wait()


def _make_sc_kernel():
    mesh = plsc.VectorSubcoreMesh(core_axis_name="c", subcore_axis_name="s")
    return pl.kernel(
        _sc_body,
        out_type=jax.ShapeDtypeStruct((128, PAD), jnp.float32),
        mesh=mesh,
        compiler_params=pltpu.CompilerParams(needs_layout_passes=False),
        scratch_types=[
            pltpu.VMEM((CHUNK_ROWS, W), jnp.float32),
            pltpu.VMEM((CHUNK_ROWS, W), jnp.float32),
            pltpu.VMEM((CHUNK_ROWS, W), jnp.float32),
            pltpu.VMEM((PAD,), jnp.int32),
            pltpu.VMEM((PAD,), jnp.int32),
            pltpu.VMEM((PAD,), jnp.float32),
            pltpu.VMEM((PAD,), jnp.float32),
            pltpu.VMEM((PAD,), jnp.float32),
            pltpu.VMEM((PAD,), jnp.float32),
            pltpu.SemaphoreType.DMA,
            pltpu.SemaphoreType.DMA,
            pltpu.SemaphoreType.DMA,
            pltpu.SemaphoreType.DMA,
        ],
    )


def _loss_body(x_ref, i0_ref, i1_ref, o_ref):
    m0 = jnp.broadcast_to((i0_ref[...] >> 9) < HALF_ROWS, (16, PAD))
    m1 = jnp.broadcast_to((i1_ref[...] >> 9) < HALF_ROWS, (16, PAD))
    p0 = jnp.where(m0, x_ref[0:16], x_ref[16:32])
    p1 = jnp.where(m1, x_ref[32:48], x_ref[48:64])
    g0 = jnp.where(m0, x_ref[64:80], x_ref[80:96])
    g1 = jnp.where(m1, x_ref[96:112], x_ref[112:128])
    d = p0 - p1
    g = g0 - g1
    sq = d * d
    hinge = jnp.maximum(-d * jnp.sign(g), 0.0)
    loss = jnp.where(jnp.abs(g) < DELTA, sq, hinge)
    col = lax.broadcasted_iota(jnp.int32, (16, PAD), 1)
    loss = jnp.where(col < SAMPLE_SIZE, loss, 0.0)
    o_ref[0, 0] = jnp.sum(loss) * (1.0 / (SAMPLE_SIZE * B))


_loss = pl.pallas_call(
    _loss_body,
    out_shape=jax.ShapeDtypeStruct((1, 1), jnp.float32),
    out_specs=pl.BlockSpec(memory_space=pltpu.SMEM),
)


@jax.jit
def kernel(depth_pred, depth_gt, indices):
    dp2 = depth_pred.reshape(B * H, W)
    dg2 = depth_gt.reshape(B * H, W)
    i0 = jnp.pad(indices[:, 0], (0, PAD - SAMPLE_SIZE))
    i1 = jnp.pad(indices[:, 1], (0, PAD - SAMPLE_SIZE))
    vals = _make_sc_kernel()(dp2, dg2, i0, i1)
    return _loss(vals, i0, i1)[0, 0]
